# Initial kernel scaffold; baseline (speedup 1.0000x reference)
import jax
import jax.numpy as jnp
from jax.experimental import pallas as pl


def kernel(rho):
    raise NotImplementedError("write your pallas kernel here")


# speedup vs baseline: 559.3545x; 559.3545x over previous
import jax
import jax.numpy as jnp
from jax.experimental import pallas as pl

_B = 64
_N_IN = 512
_N_OUT = 1024


def _insert_kernel(rho_ref, out_ref):
    out_ref[0, :_N_IN, :] = jnp.zeros((_N_IN, _N_OUT), jnp.float32)
    out_ref[0, _N_IN:, :_N_IN] = jnp.zeros((_N_IN, _N_IN), jnp.float32)
    out_ref[0, _N_IN:, _N_IN:] = rho_ref[0]


def kernel(rho):
    return pl.pallas_call(
        _insert_kernel,
        grid=(_B,),
        in_specs=[pl.BlockSpec((1, _N_IN, _N_IN), lambda b: (b, 0, 0))],
        out_specs=pl.BlockSpec((1, _N_OUT, _N_OUT), lambda b: (b, 0, 0)),
        out_shape=jax.ShapeDtypeStruct((_B, _N_OUT, _N_OUT), jnp.float32),
    )(rho)
